# baseline (speedup 1.0000x reference)
import functools

import jax
import jax.numpy as jnp
from jax import lax
from jax.experimental import pallas as pl
from jax.experimental.pallas import tpu as pltpu
from jax.experimental.pallas import tpu_sc as plsc

NUM_HEADS = 8
HEAD_DIM = 16
NEG_SLOPE = 0.2

NSUB = 16
LANES = 16

EDGE_CHUNK = 1000
ZROWS = 500


def _take16(v, idx):
  dnums = lax.GatherDimensionNumbers(
      offset_dims=(), collapsed_slice_dims=(0,), start_index_map=(0,))
  return lax.gather(
      v, idx.reshape(16, 1), dnums, (1,),
      mode=lax.GatherScatterMode.PROMISE_IN_BOUNDS)


def _stage_a_lr(features, Wstack, Astack):
  N, IN_DIM = features.shape
  R = 5000
  NB = N // R

  def body(x_ref, w_ref, a_ref, lr_ref):
    wa = jnp.dot(w_ref[0], a_ref[0], preferred_element_type=jnp.float32)
    lr_ref[0] = jnp.dot(x_ref[...], wa, preferred_element_type=jnp.float32)

  return pl.pallas_call(
      body,
      grid=(2, NB),
      in_specs=[
          pl.BlockSpec((R, IN_DIM), lambda c, i: (i, 0)),
          pl.BlockSpec((1, IN_DIM, 128), lambda c, i: (c, 0, 0)),
          pl.BlockSpec((1, IN_DIM, 16), lambda c, i: (c, 0, 0)),
      ],
      out_specs=pl.BlockSpec((1, R, 16), lambda c, i: (c, i, 0)),
      out_shape=jax.ShapeDtypeStruct((2, N, 16), jnp.float32),
  )(features, Wstack, Astack)


def _stage_a_feat(features, Wstack):
  N, IN_DIM = features.shape
  R = 5000
  NB = N // R

  def body(x_ref, w_ref, f_ref, feat_s):
    hg = pl.program_id(2)

    @pl.when(hg == 0)
    def _():
      feat_s[...] = jnp.dot(x_ref[...], w_ref[0],
                            preferred_element_type=jnp.float32)

    for k in range(4):
      @pl.when(hg == k)
      def _():
        f_ref[0] = feat_s[:, 32 * k:32 * (k + 1)]

  return pl.pallas_call(
      body,
      grid=(2, NB, 4),
      in_specs=[
          pl.BlockSpec((R, IN_DIM), lambda c, i, hg: (i, 0)),
          pl.BlockSpec((1, IN_DIM, 128), lambda c, i, hg: (c, 0, 0)),
      ],
      out_specs=pl.BlockSpec((1, R, 32), lambda c, i, hg: (c * 4 + hg, i, 0)),
      out_shape=jax.ShapeDtypeStruct((8, N, 32), jnp.float32),
      scratch_shapes=[pltpu.VMEM((R, 128), jnp.float32)],
  )(features, Wstack)


def _stage_b1(lr, edges5, N, E):
  ept = E // NSUB
  B = 500
  G = 2
  ngrp = ept // (G * B)
  nchg = ngrp // 2
  rpt = N // NSUB

  mesh = plsc.VectorSubcoreMesh(core_axis_name="c", subcore_axis_name="s")

  @functools.partial(
      pl.kernel,
      out_type=(
          jax.ShapeDtypeStruct((2, N, 16), jnp.float32),
          jax.ShapeDtypeStruct((2, E // 2, 16), jnp.float32),
      ),
      mesh=mesh,
      compiler_params=pltpu.CompilerParams(use_tc_tiling_on_sc=False),
      scratch_types=[
          pltpu.VMEM((G, B), jnp.int32),
          pltpu.VMEM((G, B), jnp.int32),
          pltpu.VMEM((G, B), jnp.int32),
          pltpu.VMEM((G, B), jnp.int32),
          pltpu.VMEM((B, 16), jnp.float32),
          pltpu.VMEM((B, 16), jnp.float32),
          pltpu.VMEM((B, 16), jnp.float32),
          pltpu.VMEM((B, 16), jnp.float32),
          pltpu.VMEM((B, 16), jnp.float32),
          pltpu.VMEM((B // 2, 16), jnp.float32),
          pltpu.VMEM_SHARED((N, 16), jnp.float32),
          pltpu.SemaphoreType.DMA,
          pltpu.SemaphoreType.DMA,
          pltpu.SemaphoreType.DMA,
          pltpu.SemaphoreType.DMA,
      ],
  )
  def sc_b1(lr_h, edges_h, den_h, ex_h,
            src0_v, src1_v, dst0_v, dst1_v, lrs0_v, lrs1_v, lrd0_v, lrd1_v,
            exb_v, exp_v, acc_s, isem0, isem1, gsem0, gsem1):
    c = lax.axis_index("c")
    s = lax.axis_index("s")
    ebase = s * ept
    rbase = s * rpt
    zero16 = jnp.zeros((16,), jnp.float32)
    lane = lax.iota(jnp.int32, 16)
    low7 = jnp.bitwise_and(lane, 7)
    rot = low7 + 8
    shift = low7
    low8 = lane < 8
    srcb = (src0_v, src1_v)
    dstb = (dst0_v, dst1_v)
    lrsb = (lrs0_v, lrs1_v)
    lrdb = (lrd0_v, lrd1_v)
    isem = (isem0, isem1)
    gsem = (gsem0, gsem1)
    row0 = s * (ept // B)

    @pl.loop(0, B)
    def _(i):
      exb_v[i, 0:16] = zero16

    nz = rpt // B
    @pl.loop(0, nz)
    def _(j):
      pltpu.sync_copy(exb_v, acc_s.at[pl.ds(rbase + j * B, B)])
    if rpt - nz * B:
      pltpu.sync_copy(exb_v.at[pl.ds(0, rpt - nz * B)],
                      acc_s.at[pl.ds(rbase + nz * B, rpt - nz * B)])
    plsc.subcore_barrier()

    def idx_start(g, ib):
      pltpu.async_copy(edges_h.at[c, 0, pl.ds(row0 + g * G, G)],
                       srcb[ib], isem[ib])
      pltpu.async_copy(edges_h.at[c, 1, pl.ds(row0 + g * G, G)],
                       dstb[ib], isem[ib])

    def idx_wait(g, ib):
      pltpu.make_async_copy(edges_h.at[c, 0, pl.ds(row0 + g * G, G)],
                            srcb[ib], isem[ib]).wait()
      pltpu.make_async_copy(edges_h.at[c, 1, pl.ds(row0 + g * G, G)],
                            dstb[ib], isem[ib]).wait()

    def gather_start(i, ib, fb):
      pltpu.async_copy(lr_h.at[c].at[srcb[ib].at[i]], lrsb[fb], gsem[fb])
      pltpu.async_copy(lr_h.at[c].at[dstb[ib].at[i]], lrdb[fb], gsem[fb])

    def gather_wait(i, ib, fb):
      pltpu.make_async_copy(lr_h.at[c].at[srcb[ib].at[i]], lrsb[fb],
                            gsem[fb]).wait()
      pltpu.make_async_copy(lr_h.at[c].at[dstb[ib].at[i]], lrdb[fb],
                            gsem[fb]).wait()

    idx_start(0, 0)
    idx_wait(0, 0)
    gather_start(0, 0, 0)

    def chunk_body(g, i, ib, last):
      k = g * G + i
      fb = i
      nfb = 1 - fb
      if i < G - 1:
        gather_start(i + 1, ib, nfb)
      elif not last:
        idx_wait(g + 1, 1 - ib)
        gather_start(0, 1 - ib, nfb)
      gather_wait(i, ib, fb)
      lrs_v = lrsb[fb]
      lrd_v = lrdb[fb]

      @pl.loop(0, B, step=2, unroll=4)
      def _(e):
        a0 = lrs_v[e]
        b0 = lrd_v[e]
        a1 = lrs_v[e + 1]
        b1 = lrd_v[e + 1]
        s0 = a0 + _take16(b0, rot)
        s1 = a1 + _take16(b1, rot)
        e0 = jnp.exp(jnp.maximum(s0, NEG_SLOPE * s0))
        e1 = jnp.exp(jnp.maximum(s1, NEG_SLOPE * s1))
        e0 = jnp.where(low8, e0, 0.0)
        exb_v[e, 0:16] = e0
        exb_v[e + 1, 0:16] = jnp.where(low8, e1, 0.0)
        exp_v[lax.div(e, 2)] = jnp.where(low8, e0, _take16(e1, shift))

      pltpu.sync_copy(exb_v, acc_s.at[dstb[ib].at[i]], add=True)
      pltpu.sync_copy(exp_v,
                      ex_h.at[c, pl.ds((ebase + k * B) // 2, B // 2)])

    def group_body(g, ib, last):
      for i in range(G):
        chunk_body(g, i, ib, last)

    @pl.loop(0, nchg)
    def _(t):
      g0 = 2 * t
      idx_start(g0 + 1, 1)
      group_body(g0, 0, False)
      idx_start(g0 + 2, 0)
      group_body(g0 + 1, 1, False)

    group_body(ngrp - 1, 0, True)

    plsc.subcore_barrier()
    pltpu.sync_copy(acc_s.at[pl.ds(rbase, rpt)],
                    den_h.at[c, pl.ds(rbase, rpt)])

  return sc_b1(lr, edges5)


def _stage_b2(feat4, ex, den, biases, edges4, N, E):
  ept = E // NSUB
  B = 200
  G = 5
  ngrp = ept // (G * B)
  nchg = ngrp // 2
  rpt = N // NSUB
  zch = 200
  DR = 125

  mesh = plsc.VectorSubcoreMesh(core_axis_name="c", subcore_axis_name="s")

  @functools.partial(
      pl.kernel,
      out_type=jax.ShapeDtypeStruct((2, N, 128), jnp.float32),
      mesh=mesh,
      compiler_params=pltpu.CompilerParams(use_tc_tiling_on_sc=False),
      scratch_types=[
          pltpu.VMEM((G, B), jnp.int32),
          pltpu.VMEM((G, B), jnp.int32),
          pltpu.VMEM((G, B), jnp.int32),
          pltpu.VMEM((G, B), jnp.int32),
          pltpu.VMEM((B // 2, 16), jnp.float32),
          pltpu.VMEM((B // 2, 16), jnp.float32),
          pltpu.VMEM((B, 32), jnp.float32),
          pltpu.VMEM((B, 32), jnp.float32),
          pltpu.VMEM((DR, 32), jnp.float32),
          pltpu.VMEM((DR, 16), jnp.float32),
          pltpu.VMEM((DR, 16), jnp.float32),
          pltpu.VMEM((128,), jnp.float32),
          pltpu.VMEM_SHARED((N, 32), jnp.float32),
          pltpu.SemaphoreType.DMA,
          pltpu.SemaphoreType.DMA,
          pltpu.SemaphoreType.DMA,
          pltpu.SemaphoreType.DMA,
          pltpu.SemaphoreType.DMA,
          pltpu.SemaphoreType.DMA,
          pltpu.SemaphoreType.DMA,
          pltpu.SemaphoreType.DMA,
      ],
  )
  def sc_b2(feat4_h, ex_h, den_h, bias_h, edges_h,
            h2_h,
            src0_v, src1_v, dst0_v, dst1_v, exp0_v, exp1_v,
            feat0_v, feat1_v, qn_v, qd0_v, qd1_v, bias_v, acc_s,
            isem0, isem1, fsem0, fsem1, esem0, esem1, dsem0, dsem1):
    c = lax.axis_index("c")
    s = lax.axis_index("s")
    rbase = s * rpt
    zero16 = jnp.zeros((16,), jnp.float32)
    srcb = (src0_v, src1_v)
    dstb = (dst0_v, dst1_v)
    expb = (exp0_v, exp1_v)
    featb = (feat0_v, feat1_v)
    isem = (isem0, isem1)
    fsem = (fsem0, fsem1)
    esem = (esem0, esem1)
    qdb = (qd0_v, qd1_v)
    dsem = (dsem0, dsem1)
    row0 = s * (ept // B)

    pltpu.sync_copy(bias_h.at[c], bias_v)

    def idx_start(g, ib):
      pltpu.async_copy(edges_h.at[c, 0, pl.ds(row0 + g * G, G)],
                       srcb[ib], isem[ib])
      pltpu.async_copy(edges_h.at[c, 1, pl.ds(row0 + g * G, G)],
                       dstb[ib], isem[ib])

    def idx_wait(g, ib):
      pltpu.make_async_copy(edges_h.at[c, 0, pl.ds(row0 + g * G, G)],
                            srcb[ib], isem[ib]).wait()
      pltpu.make_async_copy(edges_h.at[c, 1, pl.ds(row0 + g * G, G)],
                            dstb[ib], isem[ib]).wait()

    def gather_start(p, k, i, ib, fb):
      pltpu.async_copy(feat4_h.at[c * 4 + p].at[srcb[ib].at[i]],
                       featb[fb], fsem[fb])
      ebase2 = (s * ept + k * B) // 2
      pltpu.async_copy(ex_h.at[c, pl.ds(ebase2, B // 2)], expb[fb], esem[fb])

    def gather_wait(p, k, i, ib, fb):
      pltpu.make_async_copy(feat4_h.at[c * 4 + p].at[srcb[ib].at[i]],
                            featb[fb], fsem[fb]).wait()
      ebase2 = (s * ept + k * B) // 2
      pltpu.make_async_copy(ex_h.at[c, pl.ds(ebase2, B // 2)],
                            expb[fb], esem[fb]).wait()

    def compute_scatter(h0, h1, h2, h3, i, ib, fb):
      feat_v = featb[fb]
      exp_v = expb[fb]

      @pl.loop(0, B // 2, unroll=4)
      def _(j):
        row = exp_v[j]
        e = 2 * j
        feat_v[e, 0:16] = feat_v[e, 0:16] * _take16(row, h0)
        feat_v[e, 16:32] = feat_v[e, 16:32] * _take16(row, h1)
        feat_v[e + 1, 0:16] = feat_v[e + 1, 0:16] * _take16(row, h2)
        feat_v[e + 1, 16:32] = feat_v[e + 1, 16:32] * _take16(row, h3)

      pltpu.sync_copy(feat_v, acc_s.at[dstb[ib].at[i]], add=True)

    for p in range(4):
      h0 = jnp.full((16,), 2 * p, jnp.int32)
      h1 = jnp.full((16,), 2 * p + 1, jnp.int32)
      h2 = jnp.full((16,), 8 + 2 * p, jnp.int32)
      h3 = jnp.full((16,), 8 + 2 * p + 1, jnp.int32)
      bias0 = bias_v[pl.ds(32 * p, 16)]
      bias1 = bias_v[pl.ds(32 * p + 16, 16)]

      if p == 0:
        @pl.loop(0, B)
        def _(i):
          feat0_v[i, 0:16] = zero16
          feat0_v[i, 16:32] = zero16

        nz = rpt // zch

        @pl.loop(0, nz)
        def _(j):
          pltpu.sync_copy(feat0_v.at[pl.ds(0, zch)],
                          acc_s.at[pl.ds(rbase + j * zch, zch)])
        if rpt - nz * zch:
          pltpu.sync_copy(feat0_v.at[pl.ds(0, rpt - nz * zch)],
                          acc_s.at[pl.ds(rbase + nz * zch, rpt - nz * zch)])
      plsc.subcore_barrier()

      idx_start(0, 0)
      idx_wait(0, 0)
      gather_start(p, 0, 0, 0, 0)

      def group_body(g, ib, last):
        for i in range(G):
          k = g * G + i
          fb = (ib + i) % 2
          nfb = 1 - fb
          if i < G - 1:
            gather_start(p, k + 1, i + 1, ib, nfb)
          elif not last:
            idx_wait(g + 1, 1 - ib)
            gather_start(p, k + 1, 0, 1 - ib, nfb)
          gather_wait(p, k, i, ib, fb)
          compute_scatter(h0, h1, h2, h3, i, ib, fb)

      @pl.loop(0, nchg)
      def _(t):
        g0 = 2 * t
        idx_start(g0 + 1, 1)
        group_body(g0, 0, False)
        idx_start(g0 + 2, 0)
        group_body(g0 + 1, 1, False)

      group_body(ngrp - 1, 0, True)

      plsc.subcore_barrier()

      hh0 = jnp.full((16,), 2 * p, jnp.int32)
      hh1 = jnp.full((16,), 2 * p + 1, jnp.int32)
      fold = p < 3
      if fold:
        @pl.loop(0, DR)
        def _(i):
          feat0_v[i, 0:16] = zero16
          feat0_v[i, 16:32] = zero16

      def dstart(t, db):
        pltpu.async_copy(den_h.at[c, pl.ds(rbase + t * DR, DR)],
                         qdb[db], dsem[db])

      def dchunk(t, db):
        r = rbase + t * DR
        pltpu.sync_copy(acc_s.at[pl.ds(r, DR)], qn_v)
        if fold:
          pltpu.sync_copy(feat0_v.at[pl.ds(0, DR)], acc_s.at[pl.ds(r, DR)])
        pltpu.make_async_copy(den_h.at[c, pl.ds(r, DR)],
                              qdb[db], dsem[db]).wait()
        qd_v = qdb[db]

        @pl.loop(0, DR, unroll=1)
        def _(j):
          d = qd_v[j]
          rec = jnp.where(d > 0.0, 1.0 / d, 0.0)
          qn_v[j, 0:16] = qn_v[j, 0:16] * _take16(rec, hh0) + bias0
          qn_v[j, 16:32] = qn_v[j, 16:32] * _take16(rec, hh1) + bias1

        pltpu.sync_copy(qn_v,
                        h2_h.at[c, pl.ds(r, DR), pl.ds(32 * p, 32)])

      nDR = rpt // DR
      dstart(0, 0)

      @pl.loop(0, nDR // 2)
      def _(u):
        t = 2 * u
        dstart(t + 1, 1)
        dchunk(t, 0)
        dstart(t + 2, 0)
        dchunk(t + 1, 1)

      dchunk(nDR - 1, 0)

      plsc.subcore_barrier()

  return sc_b2(feat4, ex, den, biases, edges4)


def _stage_c(h2, W1, b1, W2, b2):
  N = h2.shape[1]
  R = 5000
  NB = N // R

  def body(hp, hn, w1_ref, b1_ref, w2_ref, b2_ref, hp_ref, hn_ref, hf_ref):
    hpb = hp[0]
    hnb = hn[0]
    hp_ref[...] = hpb
    hn_ref[...] = hnb
    h = jnp.concatenate([hpb, hnb], axis=1)
    z = jnp.maximum(jnp.dot(h, w1_ref[...], preferred_element_type=jnp.float32)
                    + b1_ref[...], 0.0)
    hf_ref[...] = jnp.dot(z, w2_ref[...],
                          preferred_element_type=jnp.float32) + b2_ref[...]

  return pl.pallas_call(
      body,
      grid=(NB,),
      in_specs=[
          pl.BlockSpec((1, R, 128), lambda i: (0, i, 0)),
          pl.BlockSpec((1, R, 128), lambda i: (1, i, 0)),
          pl.BlockSpec((256, 128), lambda i: (0, 0)),
          pl.BlockSpec((1, 128), lambda i: (0, 0)),
          pl.BlockSpec((128, 128), lambda i: (0, 0)),
          pl.BlockSpec((1, 128), lambda i: (0, 0)),
      ],
      out_specs=[pl.BlockSpec((R, 128), lambda i: (i, 0))] * 3,
      out_shape=[jax.ShapeDtypeStruct((N, 128), jnp.float32)] * 3,
  )(h2, h2, W1, b1.reshape(1, 128), W2, b2.reshape(1, 128))


def kernel(features, pos_edge_index, neg_edge_index, pos_W, pos_attn_l,
           pos_attn_r, pos_bias, neg_W, neg_attn_l, neg_attn_r, neg_bias,
           W1, b1, W2, b2):
  N = features.shape[0]
  E = pos_edge_index.shape[1]

  Wstack = jnp.stack([pos_W, neg_W])
  eye8 = jnp.eye(NUM_HEADS, dtype=jnp.float32)

  def build_A(attn_l, attn_r):
    Al = (attn_l[:, :, None] * eye8[:, None, :]).reshape(128, NUM_HEADS)
    Ar = (attn_r[:, :, None] * eye8[:, None, :]).reshape(128, NUM_HEADS)
    return jnp.concatenate([Al, Ar], axis=1)

  Astack = jnp.stack([build_A(pos_attn_l, pos_attn_r),
                      build_A(neg_attn_l, neg_attn_r)])
  biases = jnp.stack([pos_bias, neg_bias])
  edges = jnp.stack([pos_edge_index, neg_edge_index]).astype(jnp.int32)

  lr = _stage_a_lr(features, Wstack, Astack)
  feat4 = _stage_a_feat(features, Wstack)
  edges5 = edges.reshape(2, 2, E // 500, 500)
  den, ex = _stage_b1(lr, edges5, N, E)
  edges4 = edges.reshape(2, 2, E // 200, 200)
  h2 = _stage_b2(feat4, ex, den, biases, edges4, N, E)
  h_pos, h_neg, h_final = _stage_c(h2, W1, b1, W2, b2)
  return (h_pos, h_neg, h_final)


# speedup vs baseline: 1.0155x; 1.0155x over previous
import functools

import jax
import jax.numpy as jnp
from jax import lax
from jax.experimental import pallas as pl
from jax.experimental.pallas import tpu as pltpu
from jax.experimental.pallas import tpu_sc as plsc

NUM_HEADS = 8
HEAD_DIM = 16
NEG_SLOPE = 0.2

NSUB = 16
LANES = 16

EDGE_CHUNK = 1000
ZROWS = 500


def _take16(v, idx):
  dnums = lax.GatherDimensionNumbers(
      offset_dims=(), collapsed_slice_dims=(0,), start_index_map=(0,))
  return lax.gather(
      v, idx.reshape(16, 1), dnums, (1,),
      mode=lax.GatherScatterMode.PROMISE_IN_BOUNDS)


def _stage_a_lr(features, Wstack, Astack):
  N, IN_DIM = features.shape
  R = 5000
  NB = N // R

  def body(x_ref, w_ref, a_ref, lr_ref):
    wa = jnp.dot(w_ref[0], a_ref[0], preferred_element_type=jnp.float32)
    lr_ref[0] = jnp.dot(x_ref[...], wa, preferred_element_type=jnp.float32)

  return pl.pallas_call(
      body,
      grid=(2, NB),
      in_specs=[
          pl.BlockSpec((R, IN_DIM), lambda c, i: (i, 0)),
          pl.BlockSpec((1, IN_DIM, 128), lambda c, i: (c, 0, 0)),
          pl.BlockSpec((1, IN_DIM, 16), lambda c, i: (c, 0, 0)),
      ],
      out_specs=pl.BlockSpec((1, R, 16), lambda c, i: (c, i, 0)),
      out_shape=jax.ShapeDtypeStruct((2, N, 16), jnp.float32),
  )(features, Wstack, Astack)


def _stage_a_feat(features, Wstack):
  N, IN_DIM = features.shape
  R = 5000
  NB = N // R

  def body(x_ref, w_ref, f0, f1, f2, f3):
    feat = jnp.dot(x_ref[...], w_ref[0], preferred_element_type=jnp.float32)
    f0[0] = feat[:, 0:32]
    f1[0] = feat[:, 32:64]
    f2[0] = feat[:, 64:96]
    f3[0] = feat[:, 96:128]

  return pl.pallas_call(
      body,
      grid=(2, NB),
      in_specs=[
          pl.BlockSpec((R, IN_DIM), lambda c, i: (i, 0)),
          pl.BlockSpec((1, IN_DIM, 128), lambda c, i: (c, 0, 0)),
      ],
      out_specs=[pl.BlockSpec((1, R, 32), lambda c, i: (c, i, 0))
                 for _ in range(4)],
      out_shape=[jax.ShapeDtypeStruct((2, N, 32), jnp.float32)
                 for _ in range(4)],
  )(features, Wstack)


def _stage_b1(lr, edges5, N, E):
  ept = E // NSUB
  B = 500
  G = 2
  ngrp = ept // (G * B)
  nchg = ngrp // 2
  rpt = N // NSUB

  mesh = plsc.VectorSubcoreMesh(core_axis_name="c", subcore_axis_name="s")

  @functools.partial(
      pl.kernel,
      out_type=(
          jax.ShapeDtypeStruct((2, N, 16), jnp.float32),
          jax.ShapeDtypeStruct((2, E // 2, 16), jnp.float32),
      ),
      mesh=mesh,
      compiler_params=pltpu.CompilerParams(use_tc_tiling_on_sc=False),
      scratch_types=[
          pltpu.VMEM((G, B), jnp.int32),
          pltpu.VMEM((G, B), jnp.int32),
          pltpu.VMEM((G, B), jnp.int32),
          pltpu.VMEM((G, B), jnp.int32),
          pltpu.VMEM((B, 16), jnp.float32),
          pltpu.VMEM((B, 16), jnp.float32),
          pltpu.VMEM((B, 16), jnp.float32),
          pltpu.VMEM((B, 16), jnp.float32),
          pltpu.VMEM((B, 16), jnp.float32),
          pltpu.VMEM((B // 2, 16), jnp.float32),
          pltpu.VMEM_SHARED((N, 16), jnp.float32),
          pltpu.SemaphoreType.DMA,
          pltpu.SemaphoreType.DMA,
          pltpu.SemaphoreType.DMA,
          pltpu.SemaphoreType.DMA,
      ],
  )
  def sc_b1(lr_h, edges_h, den_h, ex_h,
            src0_v, src1_v, dst0_v, dst1_v, lrs0_v, lrs1_v, lrd0_v, lrd1_v,
            exb_v, exp_v, acc_s, isem0, isem1, gsem0, gsem1):
    c = lax.axis_index("c")
    s = lax.axis_index("s")
    ebase = s * ept
    rbase = s * rpt
    zero16 = jnp.zeros((16,), jnp.float32)
    lane = lax.iota(jnp.int32, 16)
    low7 = jnp.bitwise_and(lane, 7)
    rot = low7 + 8
    shift = low7
    low8 = lane < 8
    srcb = (src0_v, src1_v)
    dstb = (dst0_v, dst1_v)
    lrsb = (lrs0_v, lrs1_v)
    lrdb = (lrd0_v, lrd1_v)
    isem = (isem0, isem1)
    gsem = (gsem0, gsem1)
    row0 = s * (ept // B)

    @pl.loop(0, B)
    def _(i):
      exb_v[i, 0:16] = zero16

    nz = rpt // B
    @pl.loop(0, nz)
    def _(j):
      pltpu.sync_copy(exb_v, acc_s.at[pl.ds(rbase + j * B, B)])
    if rpt - nz * B:
      pltpu.sync_copy(exb_v.at[pl.ds(0, rpt - nz * B)],
                      acc_s.at[pl.ds(rbase + nz * B, rpt - nz * B)])
    plsc.subcore_barrier()

    def idx_start(g, ib):
      pltpu.async_copy(edges_h.at[c, 0, pl.ds(row0 + g * G, G)],
                       srcb[ib], isem[ib])
      pltpu.async_copy(edges_h.at[c, 1, pl.ds(row0 + g * G, G)],
                       dstb[ib], isem[ib])

    def idx_wait(g, ib):
      pltpu.make_async_copy(edges_h.at[c, 0, pl.ds(row0 + g * G, G)],
                            srcb[ib], isem[ib]).wait()
      pltpu.make_async_copy(edges_h.at[c, 1, pl.ds(row0 + g * G, G)],
                            dstb[ib], isem[ib]).wait()

    def gather_start(i, ib, fb):
      pltpu.async_copy(lr_h.at[c].at[srcb[ib].at[i]], lrsb[fb], gsem[fb])
      pltpu.async_copy(lr_h.at[c].at[dstb[ib].at[i]], lrdb[fb], gsem[fb])

    def gather_wait(i, ib, fb):
      pltpu.make_async_copy(lr_h.at[c].at[srcb[ib].at[i]], lrsb[fb],
                            gsem[fb]).wait()
      pltpu.make_async_copy(lr_h.at[c].at[dstb[ib].at[i]], lrdb[fb],
                            gsem[fb]).wait()

    idx_start(0, 0)
    idx_wait(0, 0)
    gather_start(0, 0, 0)

    def chunk_body(g, i, ib, last):
      k = g * G + i
      fb = i
      nfb = 1 - fb
      if i < G - 1:
        gather_start(i + 1, ib, nfb)
      elif not last:
        idx_wait(g + 1, 1 - ib)
        gather_start(0, 1 - ib, nfb)
      gather_wait(i, ib, fb)
      lrs_v = lrsb[fb]
      lrd_v = lrdb[fb]

      @pl.loop(0, B, step=2, unroll=4)
      def _(e):
        a0 = lrs_v[e]
        b0 = lrd_v[e]
        a1 = lrs_v[e + 1]
        b1 = lrd_v[e + 1]
        s0 = a0 + _take16(b0, rot)
        s1 = a1 + _take16(b1, rot)
        e0 = jnp.exp(jnp.maximum(s0, NEG_SLOPE * s0))
        e1 = jnp.exp(jnp.maximum(s1, NEG_SLOPE * s1))
        e0 = jnp.where(low8, e0, 0.0)
        exb_v[e, 0:16] = e0
        exb_v[e + 1, 0:16] = jnp.where(low8, e1, 0.0)
        exp_v[lax.div(e, 2)] = jnp.where(low8, e0, _take16(e1, shift))

      pltpu.sync_copy(exb_v, acc_s.at[dstb[ib].at[i]], add=True)
      pltpu.sync_copy(exp_v,
                      ex_h.at[c, pl.ds((ebase + k * B) // 2, B // 2)])

    def group_body(g, ib, last):
      for i in range(G):
        chunk_body(g, i, ib, last)

    @pl.loop(0, nchg)
    def _(t):
      g0 = 2 * t
      idx_start(g0 + 1, 1)
      group_body(g0, 0, False)
      idx_start(g0 + 2, 0)
      group_body(g0 + 1, 1, False)

    group_body(ngrp - 1, 0, True)

    plsc.subcore_barrier()
    pltpu.sync_copy(acc_s.at[pl.ds(rbase, rpt)],
                    den_h.at[c, pl.ds(rbase, rpt)])

  return sc_b1(lr, edges5)


def _stage_b2(ftabs, ex, den, biases, edges4, N, E):
  ept = E // NSUB
  B = 200
  G = 5
  ngrp = ept // (G * B)
  nchg = ngrp // 2
  rpt = N // NSUB
  zch = 200
  DR = 125

  mesh = plsc.VectorSubcoreMesh(core_axis_name="c", subcore_axis_name="s")

  @functools.partial(
      pl.kernel,
      out_type=jax.ShapeDtypeStruct((2, N, 128), jnp.float32),
      mesh=mesh,
      compiler_params=pltpu.CompilerParams(use_tc_tiling_on_sc=False),
      scratch_types=[
          pltpu.VMEM((G, B), jnp.int32),
          pltpu.VMEM((G, B), jnp.int32),
          pltpu.VMEM((G, B), jnp.int32),
          pltpu.VMEM((G, B), jnp.int32),
          pltpu.VMEM((B // 2, 16), jnp.float32),
          pltpu.VMEM((B // 2, 16), jnp.float32),
          pltpu.VMEM((B, 32), jnp.float32),
          pltpu.VMEM((B, 32), jnp.float32),
          pltpu.VMEM((DR, 32), jnp.float32),
          pltpu.VMEM((DR, 16), jnp.float32),
          pltpu.VMEM((DR, 16), jnp.float32),
          pltpu.VMEM((128,), jnp.float32),
          pltpu.VMEM_SHARED((N, 32), jnp.float32),
          pltpu.SemaphoreType.DMA,
          pltpu.SemaphoreType.DMA,
          pltpu.SemaphoreType.DMA,
          pltpu.SemaphoreType.DMA,
          pltpu.SemaphoreType.DMA,
          pltpu.SemaphoreType.DMA,
          pltpu.SemaphoreType.DMA,
          pltpu.SemaphoreType.DMA,
      ],
  )
  def sc_b2(f0_h, f1_h, f2_h, f3_h, ex_h, den_h, bias_h, edges_h,
            h2_h,
            src0_v, src1_v, dst0_v, dst1_v, exp0_v, exp1_v,
            feat0_v, feat1_v, qn_v, qd0_v, qd1_v, bias_v, acc_s,
            isem0, isem1, fsem0, fsem1, esem0, esem1, dsem0, dsem1):
    c = lax.axis_index("c")
    s = lax.axis_index("s")
    rbase = s * rpt
    zero16 = jnp.zeros((16,), jnp.float32)
    srcb = (src0_v, src1_v)
    dstb = (dst0_v, dst1_v)
    expb = (exp0_v, exp1_v)
    featb = (feat0_v, feat1_v)
    isem = (isem0, isem1)
    fsem = (fsem0, fsem1)
    esem = (esem0, esem1)
    ftab_h = (f0_h, f1_h, f2_h, f3_h)
    qdb = (qd0_v, qd1_v)
    dsem = (dsem0, dsem1)
    row0 = s * (ept // B)

    pltpu.sync_copy(bias_h.at[c], bias_v)

    def idx_start(g, ib):
      pltpu.async_copy(edges_h.at[c, 0, pl.ds(row0 + g * G, G)],
                       srcb[ib], isem[ib])
      pltpu.async_copy(edges_h.at[c, 1, pl.ds(row0 + g * G, G)],
                       dstb[ib], isem[ib])

    def idx_wait(g, ib):
      pltpu.make_async_copy(edges_h.at[c, 0, pl.ds(row0 + g * G, G)],
                            srcb[ib], isem[ib]).wait()
      pltpu.make_async_copy(edges_h.at[c, 1, pl.ds(row0 + g * G, G)],
                            dstb[ib], isem[ib]).wait()

    def gather_start(p, k, i, ib, fb):
      pltpu.async_copy(ftab_h[p].at[c].at[srcb[ib].at[i]],
                       featb[fb], fsem[fb])
      ebase2 = (s * ept + k * B) // 2
      pltpu.async_copy(ex_h.at[c, pl.ds(ebase2, B // 2)], expb[fb], esem[fb])

    def gather_wait(p, k, i, ib, fb):
      pltpu.make_async_copy(ftab_h[p].at[c].at[srcb[ib].at[i]],
                            featb[fb], fsem[fb]).wait()
      ebase2 = (s * ept + k * B) // 2
      pltpu.make_async_copy(ex_h.at[c, pl.ds(ebase2, B // 2)],
                            expb[fb], esem[fb]).wait()

    def compute_scatter(h0, h1, h2, h3, i, ib, fb):
      feat_v = featb[fb]
      exp_v = expb[fb]

      @pl.loop(0, B // 2, unroll=4)
      def _(j):
        row = exp_v[j]
        e = 2 * j
        feat_v[e, 0:16] = feat_v[e, 0:16] * _take16(row, h0)
        feat_v[e, 16:32] = feat_v[e, 16:32] * _take16(row, h1)
        feat_v[e + 1, 0:16] = feat_v[e + 1, 0:16] * _take16(row, h2)
        feat_v[e + 1, 16:32] = feat_v[e + 1, 16:32] * _take16(row, h3)

      pltpu.sync_copy(feat_v, acc_s.at[dstb[ib].at[i]], add=True)

    for p in range(4):
      h0 = jnp.full((16,), 2 * p, jnp.int32)
      h1 = jnp.full((16,), 2 * p + 1, jnp.int32)
      h2 = jnp.full((16,), 8 + 2 * p, jnp.int32)
      h3 = jnp.full((16,), 8 + 2 * p + 1, jnp.int32)
      bias0 = bias_v[pl.ds(32 * p, 16)]
      bias1 = bias_v[pl.ds(32 * p + 16, 16)]

      if p == 0:
        @pl.loop(0, B)
        def _(i):
          feat0_v[i, 0:16] = zero16
          feat0_v[i, 16:32] = zero16

        nz = rpt // zch

        @pl.loop(0, nz)
        def _(j):
          pltpu.sync_copy(feat0_v.at[pl.ds(0, zch)],
                          acc_s.at[pl.ds(rbase + j * zch, zch)])
        if rpt - nz * zch:
          pltpu.sync_copy(feat0_v.at[pl.ds(0, rpt - nz * zch)],
                          acc_s.at[pl.ds(rbase + nz * zch, rpt - nz * zch)])
      plsc.subcore_barrier()

      idx_start(0, 0)
      idx_wait(0, 0)
      gather_start(p, 0, 0, 0, 0)

      def group_body(g, ib, last):
        for i in range(G):
          k = g * G + i
          fb = (ib + i) % 2
          nfb = 1 - fb
          if i < G - 1:
            gather_start(p, k + 1, i + 1, ib, nfb)
          elif not last:
            idx_wait(g + 1, 1 - ib)
            gather_start(p, k + 1, 0, 1 - ib, nfb)
          gather_wait(p, k, i, ib, fb)
          compute_scatter(h0, h1, h2, h3, i, ib, fb)

      @pl.loop(0, nchg)
      def _(t):
        g0 = 2 * t
        idx_start(g0 + 1, 1)
        group_body(g0, 0, False)
        idx_start(g0 + 2, 0)
        group_body(g0 + 1, 1, False)

      group_body(ngrp - 1, 0, True)

      plsc.subcore_barrier()

      hh0 = jnp.full((16,), 2 * p, jnp.int32)
      hh1 = jnp.full((16,), 2 * p + 1, jnp.int32)
      fold = p < 3
      if fold:
        @pl.loop(0, DR)
        def _(i):
          feat0_v[i, 0:16] = zero16
          feat0_v[i, 16:32] = zero16

      def dstart(t, db):
        pltpu.async_copy(den_h.at[c, pl.ds(rbase + t * DR, DR)],
                         qdb[db], dsem[db])

      def dchunk(t, db):
        r = rbase + t * DR
        pltpu.sync_copy(acc_s.at[pl.ds(r, DR)], qn_v)
        if fold:
          pltpu.sync_copy(feat0_v.at[pl.ds(0, DR)], acc_s.at[pl.ds(r, DR)])
        pltpu.make_async_copy(den_h.at[c, pl.ds(r, DR)],
                              qdb[db], dsem[db]).wait()
        qd_v = qdb[db]

        @pl.loop(0, DR, unroll=1)
        def _(j):
          d = qd_v[j]
          rec = jnp.where(d > 0.0, 1.0 / d, 0.0)
          qn_v[j, 0:16] = qn_v[j, 0:16] * _take16(rec, hh0) + bias0
          qn_v[j, 16:32] = qn_v[j, 16:32] * _take16(rec, hh1) + bias1

        pltpu.sync_copy(qn_v,
                        h2_h.at[c, pl.ds(r, DR), pl.ds(32 * p, 32)])

      nDR = rpt // DR
      dstart(0, 0)

      @pl.loop(0, nDR // 2)
      def _(u):
        t = 2 * u
        dstart(t + 1, 1)
        dchunk(t, 0)
        dstart(t + 2, 0)
        dchunk(t + 1, 1)

      dchunk(nDR - 1, 0)

      plsc.subcore_barrier()

  return sc_b2(*ftabs, ex, den, biases, edges4)


def _stage_c(h2, W1, b1, W2, b2):
  N = h2.shape[1]
  R = 5000
  NB = N // R

  def body(hp, hn, w1_ref, b1_ref, w2_ref, b2_ref, hp_ref, hn_ref, hf_ref):
    hpb = hp[0]
    hnb = hn[0]
    hp_ref[...] = hpb
    hn_ref[...] = hnb
    h = jnp.concatenate([hpb, hnb], axis=1)
    z = jnp.maximum(jnp.dot(h, w1_ref[...], preferred_element_type=jnp.float32)
                    + b1_ref[...], 0.0)
    hf_ref[...] = jnp.dot(z, w2_ref[...],
                          preferred_element_type=jnp.float32) + b2_ref[...]

  return pl.pallas_call(
      body,
      grid=(NB,),
      in_specs=[
          pl.BlockSpec((1, R, 128), lambda i: (0, i, 0)),
          pl.BlockSpec((1, R, 128), lambda i: (1, i, 0)),
          pl.BlockSpec((256, 128), lambda i: (0, 0)),
          pl.BlockSpec((1, 128), lambda i: (0, 0)),
          pl.BlockSpec((128, 128), lambda i: (0, 0)),
          pl.BlockSpec((1, 128), lambda i: (0, 0)),
      ],
      out_specs=[pl.BlockSpec((R, 128), lambda i: (i, 0))] * 3,
      out_shape=[jax.ShapeDtypeStruct((N, 128), jnp.float32)] * 3,
  )(h2, h2, W1, b1.reshape(1, 128), W2, b2.reshape(1, 128))


def kernel(features, pos_edge_index, neg_edge_index, pos_W, pos_attn_l,
           pos_attn_r, pos_bias, neg_W, neg_attn_l, neg_attn_r, neg_bias,
           W1, b1, W2, b2):
  N = features.shape[0]
  E = pos_edge_index.shape[1]

  Wstack = jnp.stack([pos_W, neg_W])
  eye8 = jnp.eye(NUM_HEADS, dtype=jnp.float32)

  def build_A(attn_l, attn_r):
    Al = (attn_l[:, :, None] * eye8[:, None, :]).reshape(128, NUM_HEADS)
    Ar = (attn_r[:, :, None] * eye8[:, None, :]).reshape(128, NUM_HEADS)
    return jnp.concatenate([Al, Ar], axis=1)

  Astack = jnp.stack([build_A(pos_attn_l, pos_attn_r),
                      build_A(neg_attn_l, neg_attn_r)])
  biases = jnp.stack([pos_bias, neg_bias])
  edges = jnp.stack([pos_edge_index, neg_edge_index]).astype(jnp.int32)

  lr = _stage_a_lr(features, Wstack, Astack)
  ftabs = _stage_a_feat(features, Wstack)
  edges5 = edges.reshape(2, 2, E // 500, 500)
  den, ex = _stage_b1(lr, edges5, N, E)
  edges4 = edges.reshape(2, 2, E // 200, 200)
  h2 = _stage_b2(ftabs, ex, den, biases, edges4, N, E)
  h_pos, h_neg, h_final = _stage_c(h2, W1, b1, W2, b2)
  return (h_pos, h_neg, h_final)
